# baseline (speedup 1.0000x reference)
import functools

import jax
import jax.numpy as jnp
from jax.experimental import pallas as pl

K_WTA = 32


def _matmul_body(x_ref, w_ref, out_ref):
    out_ref[...] = jax.lax.dot_general(
        x_ref[...], w_ref[...],
        dimension_numbers=(((1,), (1,)), ((), ())),
        preferred_element_type=jnp.float32,
    )


def _topk_mask_body(x_ref, out_ref):
    x = x_ref[...]
    i = jax.lax.bitcast_convert_type(x, jnp.int32)
    key = i ^ ((i >> 31) & jnp.int32(0x7FFFFFFF))
    rows = x.shape[0]

    def count_ge(cand):
        return jnp.sum((key >= cand).astype(jnp.int32), axis=1,
                       keepdims=True).astype(jnp.float32)

    kf = jnp.float32(K_WTA)
    t = jnp.where(count_ge(jnp.int32(0)) >= kf,
                  jnp.int32(0), jnp.int32(-2147483648)).reshape(rows, 1)
    for bit in range(30, -1, -1):
        cand = t + jnp.int32(1 << bit)
        t = jnp.where(count_ge(cand) >= kf, cand, t)
    bits = key ^ ((key >> 31) & jnp.int32(0x7FFFFFFF))
    masked = jnp.where(key >= t, bits, jnp.int32(0))
    out_ref[...] = jax.lax.bitcast_convert_type(masked, jnp.float32)


@functools.partial(jax.jit, static_argnames=())
def kernel(input, W):
    B, F = input.shape
    O = W.shape[0]

    CB = 1024
    n_cb = O // CB
    x_full = pl.pallas_call(
        _matmul_body,
        grid=(n_cb,),
        in_specs=[
            pl.BlockSpec((B, F), lambda i: (0, 0)),
            pl.BlockSpec((CB, F), lambda i: (i, 0)),
        ],
        out_specs=pl.BlockSpec((B, CB), lambda i: (0, i)),
        out_shape=jax.ShapeDtypeStruct((B, O), jnp.float32),
    )(input, W)

    RB = 64
    n_rb = B // RB
    out = pl.pallas_call(
        _topk_mask_body,
        grid=(n_rb,),
        in_specs=[pl.BlockSpec((RB, O), lambda i: (i, 0))],
        out_specs=pl.BlockSpec((RB, O), lambda i: (i, 0)),
        out_shape=jax.ShapeDtypeStruct((B, O), jnp.float32),
    )(x_full)
    return out


# speedup vs baseline: 1.2286x; 1.2286x over previous
import functools

import jax
import jax.numpy as jnp
from jax.experimental import pallas as pl

K_WTA = 32


def _matmul_body(x_ref, w_ref, out_ref):
    out_ref[...] = jax.lax.dot_general(
        x_ref[...], w_ref[...],
        dimension_numbers=(((1,), (1,)), ((), ())),
        preferred_element_type=jnp.float32,
    )


def _topk_mask_body(x_ref, out_ref):
    x = x_ref[...]
    i = jax.lax.bitcast_convert_type(x, jnp.int32)
    key = i ^ ((i >> 31) & jnp.int32(0x7FFFFFFF))
    rows = x.shape[0]

    def count_ge(cand):
        return jnp.sum((key >= cand).astype(jnp.int32), axis=1,
                       keepdims=True).astype(jnp.float32)

    kf = jnp.float32(K_WTA)
    t = jnp.where(count_ge(jnp.int32(0)) >= kf,
                  jnp.int32(0), jnp.int32(-2147483648)).reshape(rows, 1)
    for bit in range(30, -1, -1):
        cand = t + jnp.int32(1 << bit)
        t = jnp.where(count_ge(cand) >= kf, cand, t)
    keep = key >= t
    out_ref[...] = jnp.where(keep, x, jnp.float32(0.0))


@functools.partial(jax.jit, static_argnames=())
def kernel(input, W):
    B, F = input.shape
    O = W.shape[0]

    CB = 1024
    n_cb = O // CB
    x_full = pl.pallas_call(
        _matmul_body,
        grid=(n_cb,),
        in_specs=[
            pl.BlockSpec((B, F), lambda i: (0, 0)),
            pl.BlockSpec((CB, F), lambda i: (i, 0)),
        ],
        out_specs=pl.BlockSpec((B, CB), lambda i: (0, i)),
        out_shape=jax.ShapeDtypeStruct((B, O), jnp.float32),
    )(input, W)

    RB = 64
    n_rb = B // RB
    out = pl.pallas_call(
        _topk_mask_body,
        grid=(n_rb,),
        in_specs=[pl.BlockSpec((RB, O), lambda i: (i, 0))],
        out_specs=pl.BlockSpec((RB, O), lambda i: (i, 0)),
        out_shape=jax.ShapeDtypeStruct((B, O), jnp.float32),
    )(x_full)
    return out


# speedup vs baseline: 1.2444x; 1.0128x over previous
import functools

import jax
import jax.numpy as jnp
from jax.experimental import pallas as pl

K_WTA = 32


def _matmul_body(x_ref, w_ref, out_ref):
    out_ref[...] = jax.lax.dot_general(
        x_ref[...], w_ref[...],
        dimension_numbers=(((1,), (1,)), ((), ())),
        preferred_element_type=jnp.float32,
    )


def _topk_mask_body(x_ref, out_ref):
    x = x_ref[...]
    i = jax.lax.bitcast_convert_type(x, jnp.int32)
    key = i ^ ((i >> 31) & jnp.int32(0x7FFFFFFF))
    rows = x.shape[0]

    def count_ge(cand):
        return jnp.sum((key >= cand).astype(jnp.int32), axis=1,
                       keepdims=True).astype(jnp.float32)

    kf = jnp.float32(K_WTA)
    t = jnp.where(count_ge(jnp.int32(0)) >= kf,
                  jnp.int32(0), jnp.int32(-2147483648)).reshape(rows, 1)
    for bit in range(30, -1, -1):
        cand = t + jnp.int32(1 << bit)
        t = jnp.where(count_ge(cand) >= kf, cand, t)
    keep = key >= t
    out_ref[...] = jnp.where(keep, x, jnp.float32(0.0))


@functools.partial(jax.jit, static_argnames=())
def kernel(input, W):
    B, F = input.shape
    O = W.shape[0]

    CB = 2048
    n_cb = O // CB
    x_full = pl.pallas_call(
        _matmul_body,
        grid=(n_cb,),
        in_specs=[
            pl.BlockSpec((B, F), lambda i: (0, 0)),
            pl.BlockSpec((CB, F), lambda i: (i, 0)),
        ],
        out_specs=pl.BlockSpec((B, CB), lambda i: (0, i)),
        out_shape=jax.ShapeDtypeStruct((B, O), jnp.float32),
    )(input, W)

    RB = 64
    n_rb = B // RB
    out = pl.pallas_call(
        _topk_mask_body,
        grid=(n_rb,),
        in_specs=[pl.BlockSpec((RB, O), lambda i: (i, 0))],
        out_specs=pl.BlockSpec((RB, O), lambda i: (i, 0)),
        out_shape=jax.ShapeDtypeStruct((B, O), jnp.float32),
    )(x_full)
    return out
